# baseline (speedup 1.0000x reference)
import functools

import jax
import jax.numpy as jnp
from jax import lax
from jax.experimental import pallas as pl
from jax.experimental.pallas import tpu as pltpu
from jax.experimental.pallas import tpu_sc as plsc

NC = 2
NS = 16
L = 16

M, D, B = 100000, 128, 16384
NB = 10
R = M // NB
RPT = 624
RLAST = R - (NS - 1) * RPT
BPT = B // NS
NV = BPT // L
C = 32
TRASH = R


def _sc_body(in_hbm, idx_hbm, val_hbm, out_hbm,
             my_idx, loc_buf, pos_buf, chunk_loc, chunk_pos, vbuf, blk, sem):
    c = lax.axis_index("c")
    s = lax.axis_index("s")

    pltpu.sync_copy(idx_hbm.at[pl.ds(s * BPT, BPT)], my_idx)

    for i in range(NB // NC):
        kb = NC * i + c
        base = kb * R

        @pl.when(s < NS - 1)
        def _():
            pltpu.sync_copy(in_hbm.at[pl.ds(base + s * RPT, RPT)],
                            blk.at[pl.ds(s * RPT, RPT)])

        @pl.when(s == NS - 1)
        def _():
            pltpu.sync_copy(in_hbm.at[pl.ds(base + (NS - 1) * RPT, RLAST)],
                            blk.at[pl.ds((NS - 1) * RPT, RLAST)])

        plsc.subcore_barrier()

        def cbody(j, cnt):
            iv = my_idx[pl.ds(j * L, L)]
            basev = jnp.full((L,), base, jnp.int32)
            limv = jnp.full((L,), base + R, jnp.int32)
            m = (iv >= basev) & (iv < limv)
            loc = iv - basev
            pos = jnp.full((L,), s * BPT + j * L, jnp.int32) + lax.iota(jnp.int32, L)
            mi = jnp.where(m, 1, 0).astype(jnp.int32)
            pc = plsc.cumsum(mi)
            dest = jnp.full((L,), cnt, jnp.int32) + pc - 1
            plsc.store_scatter(loc_buf, [dest], loc, mask=m)
            plsc.store_scatter(pos_buf, [dest], pos, mask=m)
            return cnt + jnp.sum(mi)

        cnt = lax.fori_loop(0, NV, cbody, jnp.int32(0))
        nch = (cnt + (C - 1)) // C

        def chbody(ch, carry):
            off = ch * C
            for v in range(C // L):
                lane = off + v * L + lax.iota(jnp.int32, L)
                valid = lane < cnt
                lv = loc_buf[pl.ds(off + v * L, L)]
                pv = pos_buf[pl.ds(off + v * L, L)]
                chunk_loc[pl.ds(v * L, L)] = jnp.where(valid, lv, TRASH)
                chunk_pos[pl.ds(v * L, L)] = jnp.where(valid, pv, 0)
            pltpu.async_copy(val_hbm.at[chunk_pos], vbuf, sem).wait()
            pltpu.sync_copy(vbuf, blk.at[chunk_loc], add=True)
            return carry

        lax.fori_loop(0, nch, chbody, jnp.int32(0))
        plsc.subcore_barrier()

        @pl.when(s < NS - 1)
        def _():
            pltpu.sync_copy(blk.at[pl.ds(s * RPT, RPT)],
                            out_hbm.at[pl.ds(base + s * RPT, RPT)])

        @pl.when(s == NS - 1)
        def _():
            pltpu.sync_copy(blk.at[pl.ds((NS - 1) * RPT, RLAST)],
                            out_hbm.at[pl.ds(base + (NS - 1) * RPT, RLAST)])

        plsc.subcore_barrier()


@jax.jit
def _scatter_add(input, idx32, value):
    kfn = functools.partial(
        pl.kernel,
        mesh=plsc.VectorSubcoreMesh(core_axis_name="c", subcore_axis_name="s"),
        out_type=jax.ShapeDtypeStruct((M, D), jnp.float32),
        scratch_types=[
            pltpu.VMEM((BPT,), jnp.int32),
            pltpu.VMEM((BPT + 2 * L,), jnp.int32),
            pltpu.VMEM((BPT + 2 * L,), jnp.int32),
            pltpu.VMEM((C,), jnp.int32),
            pltpu.VMEM((C,), jnp.int32),
            pltpu.VMEM((C, D), jnp.float32),
            pltpu.VMEM_SHARED((R + L, D), jnp.float32),
            pltpu.SemaphoreType.DMA,
        ],
        compiler_params=pltpu.CompilerParams(needs_layout_passes=False),
    )(_sc_body)
    return kfn(input, idx32, value)


def kernel(input, index, value):
    assert input.shape == (M, D) and value.shape == (B, D)
    return _scatter_add(input, index.astype(jnp.int32), value)
